# baseline (speedup 1.0000x reference)
import functools

import jax
import jax.numpy as jnp
from jax import lax
from jax.experimental import pallas as pl
from jax.experimental.pallas import tpu as pltpu
from jax.experimental.pallas import tpu_sc as plsc

N_CELLS = 10000
N_EDGES = 320000
OBS = 128
TW = 32
NWIN = 79
NC, NS = 2, 16
NW = NC * NS
EPW = N_EDGES // NW
CH = 400
BE = 8000


def _relu(x):
    return jnp.maximum(x, 0.0)


def _dot(a, b):
    return jax.lax.dot(a, b, preferred_element_type=jnp.float32)


def _tables_body(cells, Wab, Wd1, bd1, Wd2, bd2, Wd3, bd3,
                 S1s, S2s, S1d, S2d, tabS, tabD):
    x = cells[...]
    ab = _dot(x, Wab[...])
    h = _relu(_dot(x, Wd1[...]) + bd1[...])
    h = _relu(_dot(h, Wd2[...]) + bd2[...])
    pq = _dot(h, Wd3[...]) + bd3[...]
    tabS[...] = _dot(ab, S1s[...]) + _dot(pq, S2s[...])
    tabD[...] = _dot(ab, S1d[...]) + _dot(pq, S2d[...])


def _sc_gather_body(tabS, tabD, src, dst, gsum,
                    idx_s0, idx_s1, idx_d0, idx_d1,
                    rows_s0, rows_s1, rows_d0, rows_d1,
                    gsem0, gsem1, osem0, osem1):
    wid = lax.axis_index("s") * NC + lax.axis_index("c")
    base = wid * EPW
    idx_s = (idx_s0, idx_s1)
    idx_d = (idx_d0, idx_d1)
    rows_s = (rows_s0, rows_s1)
    rows_d = (rows_d0, rows_d1)
    gsem = (gsem0, gsem1)
    osem = (osem0, osem1)
    n = EPW // CH

    def fetch(i):
        p = i & 1
        b = base + i * CH
        pltpu.sync_copy(src.at[pl.ds(b, CH)], idx_s[p])
        pltpu.sync_copy(dst.at[pl.ds(b, CH)], idx_d[p])
        return (pltpu.async_copy(tabS.at[idx_s[p]], rows_s[p], gsem[p]),
                pltpu.async_copy(tabD.at[idx_d[p]], rows_d[p], gsem[p]))

    pend = fetch(0)
    out_pend = [None, None]
    for i in range(n):
        p = i & 1
        pend[0].wait()
        pend[1].wait()
        if i + 1 < n:
            if out_pend[1 - p] is not None:
                out_pend[1 - p].wait()
                out_pend[1 - p] = None
            pend = fetch(i + 1)

        rs, rd = rows_s[p], rows_d[p]

        @plsc.parallel_loop(0, CH, step=4)
        def _(k):
            for r in range(4):
                for h in range(TW // 16):
                    sl = pl.ds(h * 16, 16)
                    rs[k + r, sl] = rs[k + r, sl] + rd[k + r, sl]

        out_pend[p] = pltpu.async_copy(
            rs, gsum.at[pl.ds(base + i * CH, CH)], osem[p])
    for cp in out_pend:
        if cp is not None:
            cp.wait()


def _edge_body(wlo, whi, g, src3, cells, s20, s21, wc32, b1, W2p, b2, w3p, b3,
               Wc1, bc1, Wc2, bc2, Wc3, bc3,
               Wa1, ba1, Wa2, ba2, Wa3, ba3,
               Wp1c, wp1t, wp1d, bp1, Wp2, bp2, Wp3, bp3,
               pred, tot):
    i = pl.program_id(0)

    @pl.when(i == 0)
    def _():
        tot[...] = jnp.zeros_like(tot)

    a = g[...]
    pq = _dot(a, s20[...]) * _dot(a, s21[...])
    h1 = _relu(a + _dot(pq, wc32[...]) + b1[...])
    h2 = _relu(_dot(h1, W2p[...]) + b2[...])
    e = _dot(h2, w3p[...]) + b3[...]

    srcrow = src3[...].reshape(1, BE)
    lanes0 = jax.lax.broadcasted_iota(jnp.int32, (128, BE), 0)

    def win(w, _):
        m = (lanes0 + w * 128) == srcrow
        contrib = _dot(m.astype(jnp.float32), e)
        base = w * 128
        tot[pl.ds(base, 128), :] += contrib
        return _

    lax.fori_loop(wlo[i], whi[i] + 1, win, 0)

    @pl.when(i == N_EDGES // BE - 1)
    def _():
        x = cells[...]
        t = tot[pl.ds(0, N_CELLS), :]
        h = _relu(_dot(x, Wc1[...]) + bc1[...])
        h = _relu(_dot(h, Wc2[...]) + bc2[...])
        adc = _dot(h, Wc3[...]) + bc3[...]
        h = _relu(_dot(t, Wa1[...]) + ba1[...])
        h = _relu(_dot(h, Wa2[...]) + ba2[...])
        ade = _dot(h, Wa3[...]) + ba3[...]
        gg = _relu(_dot(x, Wp1c[...]) + _dot(t, wp1t[...])
                   + _dot(adc * ade, wp1d[...]) + bp1[...])
        gg = _relu(_dot(gg, Wp2[...]) + bp2[...])
        pred[...] = _dot(gg, Wp3[...]) + bp3[...]


def kernel(grid_obs, effect_inds, params):
    cells = grid_obs
    src = effect_inds[0].astype(jnp.int32)
    dst = effect_inds[1].astype(jnp.int32)
    f32 = jnp.float32

    (We1, be1), (We2, be2), (We3, be3) = params['effect']
    Wa, Wb, wc = We1[:OBS], We1[OBS:2 * OBS], We1[2 * OBS]
    edc, edn = params['effect_dotp_cell'], params['effect_dotp_neighbor']
    Wab = jnp.concatenate([Wa, Wb], axis=1)
    Wd1 = jnp.concatenate([edc[0][0], edn[0][0]], axis=1)
    bd1 = jnp.concatenate([edc[0][1], edn[0][1]])[None]
    Wd2 = jnp.zeros((40, 40), f32).at[:20, :20].set(edc[1][0]).at[20:, 20:].set(edn[1][0])
    bd2 = jnp.concatenate([edc[1][1], edn[1][1]])[None]
    Wd3 = jnp.zeros((40, 2), f32).at[:20, 0:1].set(edc[2][0]).at[20:, 1:2].set(edn[2][0])
    bd3 = jnp.concatenate([edc[2][1], edn[2][1]])[None]
    eye20 = jnp.eye(20, dtype=f32)
    S1s = jnp.zeros((40, TW), f32).at[:20, :20].set(eye20)
    S1d = jnp.zeros((40, TW), f32).at[20:, :20].set(eye20)
    S2s = jnp.zeros((2, TW), f32).at[0, 20].set(1.0)
    S2d = jnp.zeros((2, TW), f32).at[1, 21].set(1.0)

    tabS, tabD = pl.pallas_call(
        _tables_body,
        out_shape=(jax.ShapeDtypeStruct((N_CELLS, TW), f32),
                   jax.ShapeDtypeStruct((N_CELLS, TW), f32)),
    )(cells, Wab, Wd1, bd1, Wd2, bd2, Wd3, bd3, S1s, S2s, S1d, S2d)

    mesh = plsc.VectorSubcoreMesh(core_axis_name="c", subcore_axis_name="s")
    gsum = pl.kernel(
        _sc_gather_body,
        out_type=jax.ShapeDtypeStruct((N_EDGES, TW), f32),
        mesh=mesh,
        scratch_types=[pltpu.VMEM((CH,), jnp.int32),
                       pltpu.VMEM((CH,), jnp.int32),
                       pltpu.VMEM((CH,), jnp.int32),
                       pltpu.VMEM((CH,), jnp.int32),
                       pltpu.VMEM((CH, TW), f32),
                       pltpu.VMEM((CH, TW), f32),
                       pltpu.VMEM((CH, TW), f32),
                       pltpu.VMEM((CH, TW), f32),
                       pltpu.SemaphoreType.DMA,
                       pltpu.SemaphoreType.DMA,
                       pltpu.SemaphoreType.DMA,
                       pltpu.SemaphoreType.DMA],
        compiler_params=pltpu.CompilerParams(use_tc_tiling_on_sc=False),
    )(tabS, tabD, src, dst)

    s20 = jnp.zeros((TW, 1), f32).at[20, 0].set(1.0)
    s21 = jnp.zeros((TW, 1), f32).at[21, 0].set(1.0)
    wc32 = jnp.zeros((1, TW), f32).at[0, :20].set(wc)
    b1 = jnp.zeros((1, TW), f32).at[0, :20].set(be1)
    W2p = jnp.zeros((TW, TW), f32).at[:20, :20].set(We2)
    b2 = jnp.zeros((1, TW), f32).at[0, :20].set(be2)
    w3p = jnp.zeros((TW, 1), f32).at[:20].set(We3)
    b3 = be3[None]

    def _w(a):
        return pl.BlockSpec(a.shape, lambda i: (0,) * a.ndim)

    src2 = src.reshape(N_EDGES // BE, BE)
    wlo = src2[:, 0] // 128
    whi = src2[:, -1] // 128
    src3 = src.reshape(N_EDGES // BE, 1, BE)
    smem = pl.BlockSpec(memory_space=pltpu.SMEM)
    adc_p, ade_p, app_p = (params['apply_dotp_cell'],
                           params['apply_dotp_effect'], params['apply'])
    (Wp1, bp1), (Wp2, bp2), (Wp3, bp3) = app_p
    full = pl.BlockSpec((N_CELLS, OBS), lambda i: (0, 0))
    pred = pl.pallas_call(
        _edge_body,
        grid=(N_EDGES // BE,),
        in_specs=[smem, smem,
                  pl.BlockSpec((BE, TW), lambda i: (i, 0)),
                  pl.BlockSpec((1, 1, BE), lambda i: (i, 0, 0)),
                  full,
                  _w(s20), _w(s21), _w(wc32), _w(b1), _w(W2p), _w(b2),
                  _w(w3p), _w(b3)]
                 + [_w(a) for a in (
                     adc_p[0][0], adc_p[0][1][None], adc_p[1][0],
                     adc_p[1][1][None], adc_p[2][0], adc_p[2][1][None],
                     ade_p[0][0], ade_p[0][1][None], ade_p[1][0],
                     ade_p[1][1][None], ade_p[2][0], ade_p[2][1][None],
                     Wp1[:OBS], Wp1[OBS:OBS + 1], Wp1[OBS + 1:OBS + 2],
                     bp1[None], Wp2, bp2[None], Wp3, bp3[None])],
        out_specs=full,
        out_shape=jax.ShapeDtypeStruct((N_CELLS, OBS), f32),
        scratch_shapes=[pltpu.VMEM((NWIN * 128, 1), f32)],
    )(wlo, whi, gsum, src3, cells, s20, s21, wc32, b1, W2p, b2, w3p, b3,
      adc_p[0][0], adc_p[0][1][None], adc_p[1][0], adc_p[1][1][None],
      adc_p[2][0], adc_p[2][1][None],
      ade_p[0][0], ade_p[0][1][None], ade_p[1][0], ade_p[1][1][None],
      ade_p[2][0], ade_p[2][1][None],
      Wp1[:OBS], Wp1[OBS:OBS + 1], Wp1[OBS + 1:OBS + 2], bp1[None],
      Wp2, bp2[None], Wp3, bp3[None])

    return jnp.stack([pred, pred])


# speedup vs baseline: 1.2982x; 1.2982x over previous
import functools

import jax
import jax.numpy as jnp
from jax import lax
from jax.experimental import pallas as pl
from jax.experimental.pallas import tpu as pltpu
from jax.experimental.pallas import tpu_sc as plsc

N_CELLS = 10000
N_EDGES = 320000
OBS = 128
TW = 32
NWIN = 79
NC, NS = 2, 16
NW = NC * NS
EPW = N_EDGES // NW
CH = 320
BE = 8000
BE4 = BE // 4


def _relu(x):
    return jnp.maximum(x, 0.0)


def _dot(a, b):
    return jax.lax.dot(a, b, preferred_element_type=jnp.float32)


def _tables_body(cells, Wab, Wd1, bd1, Wd2, bd2, Wd3, bd3,
                 S1s, S2s, S1d, S2d, tabS, tabD):
    x = cells[...]
    ab = _dot(x, Wab[...])
    h = _relu(_dot(x, Wd1[...]) + bd1[...])
    h = _relu(_dot(h, Wd2[...]) + bd2[...])
    pq = _dot(h, Wd3[...]) + bd3[...]
    tabS[...] = _dot(ab, S1s[...]) + _dot(pq, S2s[...])
    tabD[...] = _dot(ab, S1d[...]) + _dot(pq, S2d[...])


def _sc_gather_body(tabS, tabD, src, dst, gsum,
                    idx_s0, idx_s1, idx_d0, idx_d1,
                    rows_s0, rows_s1, rows_d0, rows_d1,
                    out0, out1,
                    gsem0, gsem1, osem0, osem1):
    wid = lax.axis_index("s") * NC + lax.axis_index("c")
    idx_s = (idx_s0, idx_s1)
    idx_d = (idx_d0, idx_d1)
    rows_s = (rows_s0, rows_s1)
    rows_d = (rows_d0, rows_d1)
    outb = (out0, out1)
    gsem = (gsem0, gsem1)
    osem = (osem0, osem1)
    nch = N_EDGES // CH
    nit = (nch + NW - 1) // NW

    def fetch(j):
        p = j & 1
        c = wid + j * NW
        b = c * CH

        @pl.when(c < nch)
        def _():
            pltpu.sync_copy(src.at[pl.ds(b, CH)], idx_s[p])
            pltpu.sync_copy(dst.at[pl.ds(b, CH)], idx_d[p])
        return (pltpu.async_copy(tabS.at[idx_s[p]], rows_s[p], gsem[p]),
                pltpu.async_copy(tabD.at[idx_d[p]], rows_d[p], gsem[p]))

    pend = fetch(0)
    out_pend = [None, None]
    for j in range(nit):
        p = j & 1
        c = wid + j * NW
        pend[0].wait()
        pend[1].wait()
        if j + 1 < nit:
            if out_pend[1 - p] is not None:
                out_pend[1 - p].wait()
                out_pend[1 - p] = None
            pend = fetch(j + 1)

        rs, rd, ob = rows_s[p], rows_d[p], outb[p]

        def do_add():
            @plsc.parallel_loop(0, CH // 4, step=1)
            def _(k):
                for r in range(4):
                    for h in range(TW // 16):
                        ob[k, pl.ds(r * TW + h * 16, 16)] = (
                            rs[4 * k + r, pl.ds(h * 16, 16)]
                            + rd[4 * k + r, pl.ds(h * 16, 16)])

        if j < nit - 1:
            do_add()
            out_pend[p] = pltpu.async_copy(
                ob, gsum.at[pl.ds(c * (CH // 4), CH // 4)], osem[p])
        else:
            @pl.when(c < nch)
            def _():
                do_add()
                pltpu.sync_copy(ob, gsum.at[pl.ds(c * (CH // 4), CH // 4)])

    for cp in out_pend:
        if cp is not None:
            cp.wait()


def _edge_body(wlo, whi, g, srcT, cells, s20, s21, wc4, b1, W2p, b2, w3p, b3,
               Wc1, bc1, Wc2, bc2, Wc3, bc3,
               Wa1, ba1, Wa2, ba2, Wa3, ba3,
               Wp1c, wp1t, wp1d, bp1, Wp2, bp2, Wp3, bp3,
               pred, tot):
    i = pl.program_id(0)

    @pl.when(i == 0)
    def _():
        tot[...] = jnp.zeros_like(tot)

    a = g[...]
    pq = _dot(a, s20[...]) * _dot(a, s21[...])
    h1 = _relu(a + _dot(pq, wc4[...]) + b1[...])
    h2 = _relu(_dot(h1, W2p[...]) + b2[...])
    e4 = _dot(h2, w3p[...]) + b3[...]

    srcs = srcT[...].reshape(4, BE4)
    lanes0 = jax.lax.broadcasted_iota(jnp.int32, (128, BE4), 0)

    def win(w, _):
        lw = lanes0 + w * 128
        contrib = _dot((lw == srcs[0:1, :]).astype(jnp.float32), e4[:, 0:1])
        for s in range(1, 4):
            contrib += _dot((lw == srcs[s:s + 1, :]).astype(jnp.float32),
                            e4[:, s:s + 1])
        tot[pl.ds(w * 128, 128), :] += contrib
        return _

    lax.fori_loop(wlo[i], whi[i] + 1, win, 0)

    @pl.when(i == N_EDGES // BE - 1)
    def _():
        x = cells[...]
        t = tot[pl.ds(0, N_CELLS), :]
        h = _relu(_dot(x, Wc1[...]) + bc1[...])
        h = _relu(_dot(h, Wc2[...]) + bc2[...])
        adc = _dot(h, Wc3[...]) + bc3[...]
        h = _relu(_dot(t, Wa1[...]) + ba1[...])
        h = _relu(_dot(h, Wa2[...]) + ba2[...])
        ade = _dot(h, Wa3[...]) + ba3[...]
        gg = _relu(_dot(x, Wp1c[...]) + _dot(t, wp1t[...])
                   + _dot(adc * ade, wp1d[...]) + bp1[...])
        gg = _relu(_dot(gg, Wp2[...]) + bp2[...])
        pred[...] = _dot(gg, Wp3[...]) + bp3[...]


def kernel(grid_obs, effect_inds, params):
    cells = grid_obs
    src = effect_inds[0].astype(jnp.int32)
    dst = effect_inds[1].astype(jnp.int32)
    f32 = jnp.float32

    (We1, be1), (We2, be2), (We3, be3) = params['effect']
    Wa, Wb, wc = We1[:OBS], We1[OBS:2 * OBS], We1[2 * OBS]
    edc, edn = params['effect_dotp_cell'], params['effect_dotp_neighbor']
    Wab = jnp.concatenate([Wa, Wb], axis=1)
    Wd1 = jnp.concatenate([edc[0][0], edn[0][0]], axis=1)
    bd1 = jnp.concatenate([edc[0][1], edn[0][1]])[None]
    Wd2 = jnp.zeros((40, 40), f32).at[:20, :20].set(edc[1][0]).at[20:, 20:].set(edn[1][0])
    bd2 = jnp.concatenate([edc[1][1], edn[1][1]])[None]
    Wd3 = jnp.zeros((40, 2), f32).at[:20, 0:1].set(edc[2][0]).at[20:, 1:2].set(edn[2][0])
    bd3 = jnp.concatenate([edc[2][1], edn[2][1]])[None]
    eye20 = jnp.eye(20, dtype=f32)
    S1s = jnp.zeros((40, TW), f32).at[:20, :20].set(eye20)
    S1d = jnp.zeros((40, TW), f32).at[20:, :20].set(eye20)
    S2s = jnp.zeros((2, TW), f32).at[0, 20].set(1.0)
    S2d = jnp.zeros((2, TW), f32).at[1, 21].set(1.0)

    tabS, tabD = pl.pallas_call(
        _tables_body,
        out_shape=(jax.ShapeDtypeStruct((N_CELLS, TW), f32),
                   jax.ShapeDtypeStruct((N_CELLS, TW), f32)),
    )(cells, Wab, Wd1, bd1, Wd2, bd2, Wd3, bd3, S1s, S2s, S1d, S2d)

    mesh = plsc.VectorSubcoreMesh(core_axis_name="c", subcore_axis_name="s")
    gsum = pl.kernel(
        _sc_gather_body,
        out_type=jax.ShapeDtypeStruct((N_EDGES // 4, 128), f32),
        mesh=mesh,
        scratch_types=[pltpu.VMEM((CH,), jnp.int32),
                       pltpu.VMEM((CH,), jnp.int32),
                       pltpu.VMEM((CH,), jnp.int32),
                       pltpu.VMEM((CH,), jnp.int32),
                       pltpu.VMEM((CH, TW), f32),
                       pltpu.VMEM((CH, TW), f32),
                       pltpu.VMEM((CH, TW), f32),
                       pltpu.VMEM((CH, TW), f32),
                       pltpu.VMEM((CH // 4, 128), f32),
                       pltpu.VMEM((CH // 4, 128), f32),
                       pltpu.SemaphoreType.DMA,
                       pltpu.SemaphoreType.DMA,
                       pltpu.SemaphoreType.DMA,
                       pltpu.SemaphoreType.DMA],
        compiler_params=pltpu.CompilerParams(use_tc_tiling_on_sc=False),
    )(tabS, tabD, src, dst)

    s20 = jnp.zeros((128, 4), f32)
    s21 = jnp.zeros((128, 4), f32)
    wc4 = jnp.zeros((4, 128), f32)
    W2p = jnp.zeros((128, 128), f32)
    w3p = jnp.zeros((128, 4), f32)
    for k in range(4):
        s20 = s20.at[TW * k + 20, k].set(1.0)
        s21 = s21.at[TW * k + 21, k].set(1.0)
        wc4 = wc4.at[k, TW * k:TW * k + 20].set(wc)
        W2p = W2p.at[TW * k:TW * k + 20, TW * k:TW * k + 20].set(We2)
        w3p = w3p.at[TW * k:TW * k + 20, k].set(We3[:, 0])
    b1 = jnp.tile(jnp.zeros((1, TW), f32).at[0, :20].set(be1), (1, 4))
    b2 = jnp.tile(jnp.zeros((1, TW), f32).at[0, :20].set(be2), (1, 4))
    b3 = jnp.tile(be3[None], (1, 4))

    def _w(a):
        return pl.BlockSpec(a.shape, lambda i: (0,) * a.ndim)

    src2 = src.reshape(N_EDGES // BE, BE)
    wlo = src2[:, 0] // 128
    whi = src2[:, -1] // 128
    srcT = src.reshape(N_EDGES // BE, BE4, 4).transpose(0, 2, 1)
    smem = pl.BlockSpec(memory_space=pltpu.SMEM)
    adc_p, ade_p, app_p = (params['apply_dotp_cell'],
                           params['apply_dotp_effect'], params['apply'])
    (Wp1, bp1), (Wp2, bp2), (Wp3, bp3) = app_p
    full = pl.BlockSpec((N_CELLS, OBS), lambda i: (0, 0))
    pred = pl.pallas_call(
        _edge_body,
        grid=(N_EDGES // BE,),
        in_specs=[smem, smem,
                  pl.BlockSpec((BE4, 128), lambda i: (i, 0)),
                  pl.BlockSpec((1, 4, BE4), lambda i: (i, 0, 0)),
                  full,
                  _w(s20), _w(s21), _w(wc4), _w(b1), _w(W2p), _w(b2),
                  _w(w3p), _w(b3)]
                 + [_w(a) for a in (
                     adc_p[0][0], adc_p[0][1][None], adc_p[1][0],
                     adc_p[1][1][None], adc_p[2][0], adc_p[2][1][None],
                     ade_p[0][0], ade_p[0][1][None], ade_p[1][0],
                     ade_p[1][1][None], ade_p[2][0], ade_p[2][1][None],
                     Wp1[:OBS], Wp1[OBS:OBS + 1], Wp1[OBS + 1:OBS + 2],
                     bp1[None], Wp2, bp2[None], Wp3, bp3[None])],
        out_specs=full,
        out_shape=jax.ShapeDtypeStruct((N_CELLS, OBS), f32),
        scratch_shapes=[pltpu.VMEM((NWIN * 128, 1), f32)],
    )(wlo, whi, gsum, srcT, cells, s20, s21, wc4, b1, W2p, b2, w3p, b3,
      adc_p[0][0], adc_p[0][1][None], adc_p[1][0], adc_p[1][1][None],
      adc_p[2][0], adc_p[2][1][None],
      ade_p[0][0], ade_p[0][1][None], ade_p[1][0], ade_p[1][1][None],
      ade_p[2][0], ade_p[2][1][None],
      Wp1[:OBS], Wp1[OBS:OBS + 1], Wp1[OBS + 1:OBS + 2], bp1[None],
      Wp2, bp2[None], Wp3, bp3[None])

    return jnp.stack([pred, pred])
